# Initial kernel scaffold; baseline (speedup 1.0000x reference)
import jax
import jax.numpy as jnp
from jax.experimental import pallas as pl


def kernel(user, item, eu_gmf, ei_gmf, eu_mlp, ei_mlp, W1, b1, W2, b2, W3, b3, Wp, bp):
    raise NotImplementedError("write your pallas kernel here")


# speedup vs baseline: 1.1745x; 1.1745x over previous
import functools

import jax
import jax.numpy as jnp
from jax import lax
from jax.experimental import pallas as pl
from jax.experimental.pallas import tpu as pltpu
from jax.experimental.pallas import tpu_sc as plsc

_NC = 2
_NS = 16
_NW = _NC * _NS

_B = 16384
_CHUNK = 128
_BPW = _B // _NW
_NCHUNK = _BPW // _CHUNK


def _sc_gather_body(user_r, item_r, eu_gmf, ei_gmf, eu_mlp, ei_mlp,
                    ug_out, ig_out, um_out, im_out,
                    idx_u, idx_i, ug_v, ig_v, um_v, im_v,
                    s0, s1, s2, s3):
    wid = lax.axis_index("s") * _NC + lax.axis_index("c")
    pltpu.sync_copy(user_r.at[wid], idx_u)
    pltpu.sync_copy(item_r.at[wid], idx_i)
    base = wid * _BPW
    for j in range(_NCHUNK):
        row = base + j * _CHUNK
        c0 = pltpu.async_copy(eu_mlp.at[idx_u.at[j]], um_v, s0)
        c1 = pltpu.async_copy(ei_mlp.at[idx_i.at[j]], im_v, s1)
        c2 = pltpu.async_copy(eu_gmf.at[idx_u.at[j]], ug_v, s2)
        c3 = pltpu.async_copy(ei_gmf.at[idx_i.at[j]], ig_v, s3)
        c0.wait()
        pltpu.sync_copy(um_v, um_out.at[pl.ds(row, _CHUNK)])
        c1.wait()
        pltpu.sync_copy(im_v, im_out.at[pl.ds(row, _CHUNK)])
        c2.wait()
        pltpu.sync_copy(ug_v, ug_out.at[pl.ds(row, _CHUNK)])
        c3.wait()
        pltpu.sync_copy(ig_v, ig_out.at[pl.ds(row, _CHUNK)])


def _sc_gather(user, item, eu_gmf, ei_gmf, eu_mlp, ei_mlp):
    n_lat = eu_gmf.shape[1]
    mlp_d = eu_mlp.shape[1]
    user_r = user.astype(jnp.int32).reshape(_NW, _NCHUNK, _CHUNK)
    item_r = item.astype(jnp.int32).reshape(_NW, _NCHUNK, _CHUNK)
    mesh = plsc.VectorSubcoreMesh(core_axis_name="c", subcore_axis_name="s",
                                  num_cores=_NC, num_subcores=_NS)
    f = pl.kernel(
        _sc_gather_body,
        out_type=[
            jax.ShapeDtypeStruct((_B, n_lat), jnp.float32),
            jax.ShapeDtypeStruct((_B, n_lat), jnp.float32),
            jax.ShapeDtypeStruct((_B, mlp_d), jnp.float32),
            jax.ShapeDtypeStruct((_B, mlp_d), jnp.float32),
        ],
        mesh=mesh,
        scratch_types=[
            pltpu.VMEM((_NCHUNK, _CHUNK), jnp.int32),
            pltpu.VMEM((_NCHUNK, _CHUNK), jnp.int32),
            pltpu.VMEM((_CHUNK, n_lat), jnp.float32),
            pltpu.VMEM((_CHUNK, n_lat), jnp.float32),
            pltpu.VMEM((_CHUNK, mlp_d), jnp.float32),
            pltpu.VMEM((_CHUNK, mlp_d), jnp.float32),
            pltpu.SemaphoreType.DMA,
            pltpu.SemaphoreType.DMA,
            pltpu.SemaphoreType.DMA,
            pltpu.SemaphoreType.DMA,
        ],
        compiler_params=pltpu.CompilerParams(use_tc_tiling_on_sc=False),
    )
    return f(user_r, item_r, eu_gmf, ei_gmf, eu_mlp, ei_mlp)


def _tc_mlp_body(ug_ref, ig_ref, um_ref, im_ref,
                 w1u_ref, w1i_ref, b1_ref, w2_ref, b2_ref, w3_ref, b3_ref,
                 wp_ref, bp_ref, out_ref):
    gmf = ug_ref[...] * ig_ref[...]
    h = jnp.dot(um_ref[...], w1u_ref[...],
                preferred_element_type=jnp.float32)
    h += jnp.dot(im_ref[...], w1i_ref[...],
                 preferred_element_type=jnp.float32)
    h = jax.nn.relu(h + b1_ref[...])
    h = jax.nn.relu(jnp.dot(h, w2_ref[...],
                            preferred_element_type=jnp.float32) + b2_ref[...])
    h = jax.nn.relu(jnp.dot(h, w3_ref[...],
                            preferred_element_type=jnp.float32) + b3_ref[...])
    wp = wp_ref[...]
    n_lat = gmf.shape[1]
    acc = jnp.sum(gmf * wp[:, :n_lat], axis=1)
    acc += jnp.sum(h * wp[:, n_lat:], axis=1)
    out_ref[...] = acc + bp_ref[0]


def _tc_mlp(ug, ig, um, im, W1, b1, W2, b2, W3, b3, Wp, bp):
    n_lat = ug.shape[1]
    mlp_d = um.shape[1]
    blk = 2048
    grid = (_B // blk,)
    full = lambda shape: pl.BlockSpec(shape, lambda i: (0,) * len(shape))
    return pl.pallas_call(
        _tc_mlp_body,
        grid=grid,
        in_specs=[
            pl.BlockSpec((blk, n_lat), lambda i: (i, 0)),
            pl.BlockSpec((blk, n_lat), lambda i: (i, 0)),
            pl.BlockSpec((blk, mlp_d), lambda i: (i, 0)),
            pl.BlockSpec((blk, mlp_d), lambda i: (i, 0)),
            full((mlp_d, mlp_d)),
            full((mlp_d, mlp_d)),
            full((1, mlp_d)),
            full((mlp_d, mlp_d // 2)),
            full((1, mlp_d // 2)),
            full((mlp_d // 2, n_lat)),
            full((1, n_lat)),
            full((1, 2 * n_lat)),
            full((1, 1)),
        ],
        out_specs=pl.BlockSpec((blk,), lambda i: (i,)),
        out_shape=jax.ShapeDtypeStruct((_B,), jnp.float32),
    )(ug, ig, um, im,
      W1[:mlp_d], W1[mlp_d:], b1.reshape(1, -1),
      W2, b2.reshape(1, -1), W3, b3.reshape(1, -1),
      Wp.reshape(1, -1), bp.reshape(1, 1))


def kernel(user, item, eu_gmf, ei_gmf, eu_mlp, ei_mlp,
           W1, b1, W2, b2, W3, b3, Wp, bp):
    ug, ig, um, im = _sc_gather(user, item, eu_gmf, ei_gmf, eu_mlp, ei_mlp)
    return _tc_mlp(ug, ig, um, im, W1, b1, W2, b2, W3, b3, Wp, bp)
